# baseline (speedup 1.0000x reference)
import jax
import jax.numpy as jnp
from jax import lax
from jax.experimental import pallas as pl
from jax.experimental.pallas import tpu as pltpu
from jax.experimental.pallas import tpu_sc as plsc

_NC, _NS = 2, 16
_NW = _NC * _NS
_LANES = 16


def _sc_pool(ids, table, B, L, D, LP):
  SPLITS = ((0, 104), (104, 96))
  BPW = B // _NW
  OUTC = 64
  NOC = BPW // OUTC
  GROUP = 4
  NG = OUTC // GROUP
  ROWS = GROUP * L
  DH = D // _LANES

  mesh = plsc.VectorSubcoreMesh(core_axis_name="c", subcore_axis_name="s",
                                num_cores=_NC, num_subcores=_NS)

  def body(ids_hbm, table_hbm, out_hbm, ids_v, buf0, buf1, out_v,
           sem0, sem1):
    bufs = (buf0, buf1)
    sems = (sem0, sem1)
    wid = lax.axis_index("c") * _NS + lax.axis_index("s")

    def fire(g, b):
      descs = []
      for r in range(GROUP):
        for off, sz in SPLITS:
          irow = g * GROUP + r
          dst = bufs[b].at[pl.ds(r * L + off, sz)]
          descs.append(pltpu.async_copy(
              table_hbm.at[ids_v.at[irow, pl.ds(off, sz)]],
              dst, sems[b]))
      return descs

    def reduce_group(g, b):
      buf = bufs[b]
      zero = jnp.zeros((_LANES,), jnp.float32)

      def rbody(j, carry):
        cs = list(carry)
        for u in range(2):
          for r in range(GROUP):
            row = r * L + 2 * j + u
            for h in range(DH):
              k = r * DH + h
              cs[k] = cs[k] + buf[row, pl.ds(h * _LANES, _LANES)]
        return tuple(cs)

      acc = lax.fori_loop(0, L // 2, rbody, (zero,) * (GROUP * DH))
      for r in range(GROUP):
        for h in range(DH):
          out_v[g * GROUP + r, pl.ds(h * _LANES, _LANES)] = acc[r * DH + h]

    def oc_body(oc, carry):
      row_base = wid * BPW + oc * OUTC
      pltpu.sync_copy(ids_hbm.at[pl.ds(row_base, OUTC)], ids_v)
      descs = [None, None]
      descs[0] = fire(0, 0)
      for g in range(NG):
        b = g % 2
        if g + 1 < NG:
          descs[1 - b] = fire(g + 1, 1 - b)
        for dsc in descs[b]:
          dsc.wait()
        reduce_group(g, b)
      pltpu.sync_copy(out_v, out_hbm.at[pl.ds(wid * BPW + oc * OUTC, OUTC)])
      return carry

    lax.fori_loop(0, NOC, oc_body, 0)

  k = pl.kernel(
      body,
      out_type=jax.ShapeDtypeStruct((B, D), jnp.float32),
      mesh=mesh,
      compiler_params=pltpu.CompilerParams(use_tc_tiling_on_sc=False),
      name="sc_embed_pool",
      scratch_types=[
          pltpu.VMEM((OUTC, LP), jnp.int32),
          pltpu.VMEM((ROWS, D), jnp.float32),
          pltpu.VMEM((ROWS, D), jnp.float32),
          pltpu.VMEM((OUTC, D), jnp.float32),
          pltpu.SemaphoreType.DMA,
          pltpu.SemaphoreType.DMA,
      ],
  )
  return k(ids, table)


def _mlp(sums, input_ids, W1, b1, W2, b2):
  B, L = input_ids.shape
  D = sums.shape[1]
  H = W1.shape[0]
  T = W2.shape[0]
  BLK = 1024

  def body(sums_ref, ids_ref, W1_ref, b1_ref, W2_ref, b2_ref, out_ref):
    ids = ids_ref[...]
    cnt = jnp.sum((ids != 0).astype(jnp.float32), axis=1, keepdims=True)
    mean = sums_ref[...] / jnp.maximum(cnt, 1.0)
    h = lax.dot_general(mean, W1_ref[...], (((1,), (1,)), ((), ())),
                        preferred_element_type=jnp.float32) + b1_ref[...]
    h = jnp.maximum(h, 0.0)
    out_ref[...] = lax.dot_general(h, W2_ref[...], (((1,), (1,)), ((), ())),
                                   preferred_element_type=jnp.float32) + b2_ref[...]

  return pl.pallas_call(
      body,
      grid=(B // BLK,),
      in_specs=[
          pl.BlockSpec((BLK, D), lambda i: (i, 0)),
          pl.BlockSpec((BLK, L), lambda i: (i, 0)),
          pl.BlockSpec((H, D), lambda i: (0, 0)),
          pl.BlockSpec((1, H), lambda i: (0, 0)),
          pl.BlockSpec((T, H), lambda i: (0, 0)),
          pl.BlockSpec((1, T), lambda i: (0, 0)),
      ],
      out_specs=pl.BlockSpec((BLK, T), lambda i: (i, 0)),
      out_shape=jax.ShapeDtypeStruct((B, T), jnp.float32),
  )(sums, input_ids, W1, b1.reshape(1, H), W2, b2.reshape(1, T))


def kernel(input_ids, table, W1, b1, W2, b2):
  B, L = input_ids.shape
  D = table.shape[1]
  LP = 256
  ids_p = jnp.pad(input_ids, ((0, 0), (0, LP - L)))
  sums = _sc_pool(ids_p, table, B, L, D, LP)
  return _mlp(sums, input_ids, W1, b1, W2, b2)


# speedup vs baseline: 2.5293x; 2.5293x over previous
import jax
import jax.numpy as jnp
from jax import lax
from jax.experimental import pallas as pl
from jax.experimental.pallas import tpu as pltpu
from jax.experimental.pallas import tpu_sc as plsc

_NC, _NS = 2, 16
_NW = _NC * _NS
_LANES = 16
_PACK_N = 32768
_WPR = 16


def _sc_pool(ids, table16, B, L, D):
  SPLITS = ((0, 128), (128, 72))
  LP = 256
  BPW = B // _NW
  OUTC = 128
  NOC = BPW // OUTC
  GROUP = 8
  NG = OUTC // GROUP
  ROWS = GROUP * L
  H = _LANES

  mesh = plsc.VectorSubcoreMesh(core_axis_name="c", subcore_axis_name="s",
                                num_cores=_NC, num_subcores=_NS)

  def body(ids_hbm, table_hbm, out_hbm, ids_v, buf0, buf1, out_v,
           sem0, sem1):
    bufs = (buf0, buf1)
    sems = (sem0, sem1)
    wid = lax.axis_index("c") * _NS + lax.axis_index("s")

    def fire(g, b):
      descs = []
      for r in range(GROUP):
        for off, sz in SPLITS:
          irow = g * GROUP + r
          dst = bufs[b].at[pl.ds(r * L + off, sz)]
          descs.append(pltpu.async_copy(
              table_hbm.at[ids_v.at[irow, pl.ds(off, sz)]],
              dst, sems[b]))
      return descs

    def reduce_group(g, b):
      buf = bufs[b]
      zero = jnp.zeros((H,), jnp.float32)
      CH = 8

      def rbody(j, carry):
        cs = list(carry)
        for r in range(GROUP):
          base = r * L + CH * j
          pa = plsc.bitcast(buf[base, pl.ds(0, H)], jnp.bfloat16)
          for t in range(1, CH):
            pa = pa + plsc.bitcast(buf[base + t, pl.ds(0, H)], jnp.bfloat16)
          a, bq = plsc.unpack(pa, format=plsc.PackFormat.INTERLEAVED)
          cs[2 * r] = cs[2 * r] + a
          cs[2 * r + 1] = cs[2 * r + 1] + bq
        return tuple(cs)

      acc = lax.fori_loop(0, L // CH, rbody, (zero,) * (2 * GROUP))
      for r in range(GROUP):
        out_v[g * GROUP + r, pl.ds(0, H)] = acc[2 * r]
        out_v[g * GROUP + r, pl.ds(H, H)] = acc[2 * r + 1]

    def oc_body(oc, carry):
      row_base = wid * BPW + oc * OUTC
      pltpu.sync_copy(ids_hbm.at[pl.ds(row_base, OUTC)], ids_v)
      descs = [None, None]
      descs[0] = fire(0, 0)
      for g in range(NG):
        b = g % 2
        if g + 1 < NG:
          descs[1 - b] = fire(g + 1, 1 - b)
        for dsc in descs[b]:
          dsc.wait()
        reduce_group(g, b)
      pltpu.sync_copy(out_v, out_hbm.at[pl.ds(wid * BPW + oc * OUTC, OUTC)])
      return carry

    lax.fori_loop(0, NOC, oc_body, 0)

  k = pl.kernel(
      body,
      out_type=jax.ShapeDtypeStruct((B, D), jnp.float32),
      mesh=mesh,
      compiler_params=pltpu.CompilerParams(use_tc_tiling_on_sc=False,
                                           needs_layout_passes=False),
      name="sc_embed_pool",
      scratch_types=[
          pltpu.VMEM((OUTC, LP), jnp.int32),
          pltpu.VMEM((ROWS, _WPR), jnp.int32),
          pltpu.VMEM((ROWS, _WPR), jnp.int32),
          pltpu.VMEM((OUTC, D), jnp.float32),
          pltpu.SemaphoreType.DMA,
          pltpu.SemaphoreType.DMA,
      ],
  )
  return k(ids, table16)


def _pack_table(tableT, V, D):
  N = _PACK_N
  CH = 128 // _WPR
  C = N // CH
  G = pl.cdiv(V, N)
  HD = D // 2

  def body(tT_ref, out_ref):
    x = tT_ref[...]
    lo = lax.bitcast_convert_type(
        x[0:HD, :].astype(jnp.bfloat16), jnp.uint16).astype(jnp.uint32)
    hi = lax.bitcast_convert_type(
        x[HD:D, :].astype(jnp.bfloat16), jnp.uint16).astype(jnp.uint32)
    m = lax.bitcast_convert_type(lo | (hi << 16), jnp.int32)
    c2 = jnp.concatenate(
        [m[:, c * C:(c + 1) * C] for c in range(CH)], axis=0)
    out_ref[...] = c2.T

  return pl.pallas_call(
      body,
      grid=(G,),
      in_specs=[pl.BlockSpec((D, N), lambda i: (0, i))],
      out_specs=pl.BlockSpec((C, 128), lambda i: (i, 0)),
      out_shape=jax.ShapeDtypeStruct((G * C, 128), jnp.int32),
  )(tableT)


def _psi(v):
  N = _PACK_N
  C = N // (128 // _WPR)
  return (v & ~(N - 1)) | ((v & (C - 1)) << 3) | ((v // C) & 7)


def _mlp(sums, input_ids, W1, b1, W2, b2):
  B, L = input_ids.shape
  D = sums.shape[1]
  H = W1.shape[0]
  T = W2.shape[0]
  BLK = 1024

  def body(sums_ref, ids_ref, W1_ref, b1_ref, W2_ref, b2_ref, out_ref):
    ids = ids_ref[...]
    cnt = jnp.sum((ids != 0).astype(jnp.float32), axis=1, keepdims=True)
    mean = sums_ref[...] / jnp.maximum(cnt, 1.0)
    h = lax.dot_general(mean, W1_ref[...], (((1,), (1,)), ((), ())),
                        preferred_element_type=jnp.float32) + b1_ref[...]
    h = jnp.maximum(h, 0.0)
    out_ref[...] = lax.dot_general(h, W2_ref[...], (((1,), (1,)), ((), ())),
                                   preferred_element_type=jnp.float32) + b2_ref[...]

  return pl.pallas_call(
      body,
      grid=(B // BLK,),
      in_specs=[
          pl.BlockSpec((BLK, D), lambda i: (i, 0)),
          pl.BlockSpec((BLK, L), lambda i: (i, 0)),
          pl.BlockSpec((H, D), lambda i: (0, 0)),
          pl.BlockSpec((1, H), lambda i: (0, 0)),
          pl.BlockSpec((T, H), lambda i: (0, 0)),
          pl.BlockSpec((1, T), lambda i: (0, 0)),
      ],
      out_specs=pl.BlockSpec((BLK, T), lambda i: (i, 0)),
      out_shape=jax.ShapeDtypeStruct((B, T), jnp.float32),
  )(sums, input_ids, W1, b1.reshape(1, H), W2, b2.reshape(1, T))


def kernel(input_ids, table, W1, b1, W2, b2):
  B, L = input_ids.shape
  V, D = table.shape
  LP = 256
  ids_p = _psi(jnp.pad(input_ids, ((0, 0), (0, LP - L))))
  t128 = _pack_table(table.T, V, D)
  VP = t128.shape[0] * (128 // _WPR)
  sums = _sc_pool(ids_p, t128.reshape(VP, _WPR), B, L, D)
  return _mlp(sums, input_ids, W1, b1, W2, b2)
